# baseline (speedup 1.0000x reference)
import jax
import jax.numpy as jnp
from jax.experimental import pallas as pl
from jax.experimental.pallas import tpu as pltpu

N = 4096
M = 4096
BM = 256
GRID = M // BM
QC = 8
CW = N // QC


def _body(x_ref, y_ref, b_ref, c_ref, iy_ref, xb_ref, yb_ref,
          Q_ref, A_ref, AT_ref,
          res_ref, t1_ref, t2_ref, t3_ref,
          axp, qxp, atyp, acc_ref):
    i = pl.program_id(0)
    j = pl.program_id(1)

    xc = x_ref[...]
    yc = y_ref[...]

    ax = jnp.sum(A_ref[...] * xc, axis=1, keepdims=True)
    qx = jnp.sum(Q_ref[...] * xc, axis=1, keepdims=True)
    aty = jnp.sum(AT_ref[...] * yc, axis=1, keepdims=True)

    @pl.when(j == 0)
    def _first():
        axp[...] = ax
        qxp[...] = qx
        atyp[...] = aty

    @pl.when(j != 0)
    def _acc():
        axp[...] = axp[...] + ax
        qxp[...] = qxp[...] + qx
        atyp[...] = atyp[...] + aty

    @pl.when(j == QC - 1)
    def _stats():
        b_blk = b_ref[...]
        c_blk = c_ref[...]
        iy_blk = iy_ref[...]
        x_blk = xb_ref[...]
        y_blk = yb_ref[...]

        Ax = axp[...]
        Qx = qxp[...]
        ATy = atyp[...]

        part1 = Ax - b_blk
        part1 = part1 + iy_blk * jnp.maximum(-part1, 0.0)
        s1 = jnp.sum(part1 * part1)

        d = Qx + ATy + c_blk
        s2 = jnp.sum(d * d)

        squad = jnp.sum(x_blk * Qx)
        slin = jnp.sum(c_blk * x_blk)
        svio = jnp.sum(b_blk * y_blk)
        sb2 = jnp.sum(b_blk * b_blk)
        sc2 = jnp.sum(c_blk * c_blk)

        parts = (s1, s2, squad, slin, svio, sb2, sc2)

        @pl.when(i == 0)
        def _init():
            for k, v in enumerate(parts):
                acc_ref[k] = v

        @pl.when(i != 0)
        def _accum():
            for k, v in enumerate(parts):
                acc_ref[k] = acc_ref[k] + v

        @pl.when(i == GRID - 1)
        def _fini():
            t1 = jnp.sqrt(acc_ref[0]) / (0.0001 + jnp.sqrt(acc_ref[5]))
            t2 = jnp.sqrt(acc_ref[1]) / (0.0001 + jnp.sqrt(acc_ref[6]))
            t3 = jnp.abs(acc_ref[2] + acc_ref[3] + acc_ref[4])
            t1_ref[0, 0] = t1
            t2_ref[0, 0] = t2
            t3_ref[0, 0] = t3
            res_ref[0, 0] = t1 + t2 + t3


def kernel(Q, A, AT, b, c, x, y, Iy, il, iu, l, u):
    b2 = b[:, None]
    c2 = c[:, None]
    iy2 = Iy[:, None]
    xT = x.T
    yT = y.T

    out_shapes = [jax.ShapeDtypeStruct((1, 1), jnp.float32)] * 4
    chunk_vec = pl.BlockSpec((1, CW), lambda i, j: (0, j))
    blk_vec = pl.BlockSpec((BM, 1), lambda i, j: (i, 0))
    chunk_blk = pl.BlockSpec((BM, CW), lambda i, j: (i, j))
    scalar_out = pl.BlockSpec((1, 1), lambda i, j: (0, 0),
                              memory_space=pltpu.SMEM)

    res, t1, t2, t3 = pl.pallas_call(
        _body,
        grid=(GRID, QC),
        in_specs=[chunk_vec, chunk_vec, blk_vec, blk_vec, blk_vec, blk_vec,
                  blk_vec, chunk_blk, chunk_blk, chunk_blk],
        out_specs=[scalar_out] * 4,
        out_shape=out_shapes,
        scratch_shapes=[pltpu.VMEM((BM, 1), jnp.float32),
                        pltpu.VMEM((BM, 1), jnp.float32),
                        pltpu.VMEM((BM, 1), jnp.float32),
                        pltpu.SMEM((7,), jnp.float32)],
    )(xT, yT, b2, c2, iy2, x, y, Q, A, AT)

    return (res, t1[0, 0], t2[0, 0], t3)


# speedup vs baseline: 1.5282x; 1.5282x over previous
import jax
import jax.numpy as jnp
from jax.experimental import pallas as pl
from jax.experimental.pallas import tpu as pltpu

N = 4096
M = 4096
BM = 256
GRID = M // BM


def _body(x_ref, y_ref, b_ref, c_ref, iy_ref,
          Q_ref, A_ref, AT_ref,
          s1_ref, s2_ref, qx_ref, acc_ref):
    i = pl.program_id(0)

    x = x_ref[...]
    y = y_ref[...]
    b_blk = b_ref[...]
    c_blk = c_ref[...]
    iy_blk = iy_ref[...]

    Ax = jnp.dot(A_ref[...], x, preferred_element_type=jnp.float32)
    part1 = Ax - b_blk
    part1 = part1 + iy_blk * jnp.maximum(-part1, 0.0)
    s1 = jnp.sum(part1 * part1)

    Qx = jnp.dot(Q_ref[...], x, preferred_element_type=jnp.float32)
    ATy = jnp.dot(AT_ref[...], y, preferred_element_type=jnp.float32)
    d = Qx + ATy + c_blk
    s2 = jnp.sum(d * d)

    qx_ref[...] = Qx

    @pl.when(i == 0)
    def _init():
        acc_ref[0] = s1
        acc_ref[1] = s2

    @pl.when(i != 0)
    def _accum():
        acc_ref[0] = acc_ref[0] + s1
        acc_ref[1] = acc_ref[1] + s2

    @pl.when(i == GRID - 1)
    def _fini():
        s1_ref[0, 0] = acc_ref[0]
        s2_ref[0, 0] = acc_ref[1]


def _fini_body(xt_ref, yt_ref, bt_ref, ct_ref, x_ref, y_ref, qx_ref,
               s1_ref, s2_ref,
               res_ref, t1_ref, t2_ref, t3_ref):
    quad = jnp.dot(xt_ref[...], qx_ref[...],
                   preferred_element_type=jnp.float32)
    lin = jnp.dot(ct_ref[...], x_ref[...],
                  preferred_element_type=jnp.float32)
    vio = jnp.dot(bt_ref[...], y_ref[...],
                  preferred_element_type=jnp.float32)

    bT = bt_ref[...]
    cT = ct_ref[...]
    sb2 = jnp.sum(bT * bT)
    sc2 = jnp.sum(cT * cT)

    t1 = jnp.sqrt(s1_ref[0, 0]) / (0.0001 + jnp.sqrt(sb2))
    t2 = jnp.sqrt(s2_ref[0, 0]) / (0.0001 + jnp.sqrt(sc2))
    t3 = jnp.abs(quad[0, 0] + lin[0, 0] + vio[0, 0])
    t1_ref[0, 0] = t1
    t2_ref[0, 0] = t2
    t3_ref[0, 0] = t3
    res_ref[0, 0] = t1 + t2 + t3


def kernel(Q, A, AT, b, c, x, y, Iy, il, iu, l, u):
    b2 = b[:, None]
    c2 = c[:, None]
    iy2 = Iy[:, None]
    xT = x.T
    yT = y.T
    bT = b[None, :]
    cT = c[None, :]

    full_vec = pl.BlockSpec((N, 1), lambda i: (0, 0))
    blk_vec = pl.BlockSpec((BM, 1), lambda i: (i, 0))
    row_blk = pl.BlockSpec((BM, N), lambda i: (i, 0))
    scalar_out = pl.BlockSpec((1, 1), lambda i: (0, 0),
                              memory_space=pltpu.SMEM)

    s1, s2, qx = pl.pallas_call(
        _body,
        grid=(GRID,),
        in_specs=[full_vec, full_vec, blk_vec, blk_vec, blk_vec,
                  row_blk, row_blk, row_blk],
        out_specs=[scalar_out, scalar_out, blk_vec],
        out_shape=[jax.ShapeDtypeStruct((1, 1), jnp.float32),
                   jax.ShapeDtypeStruct((1, 1), jnp.float32),
                   jax.ShapeDtypeStruct((N, 1), jnp.float32)],
        scratch_shapes=[pltpu.SMEM((2,), jnp.float32)],
    )(x, y, b2, c2, iy2, Q, A, AT)

    scalar_in = pl.BlockSpec(memory_space=pltpu.SMEM)
    res, t1, t2, t3 = pl.pallas_call(
        _fini_body,
        in_specs=[pl.BlockSpec()] * 7 + [scalar_in] * 2,
        out_specs=[pl.BlockSpec(memory_space=pltpu.SMEM)] * 4,
        out_shape=[jax.ShapeDtypeStruct((1, 1), jnp.float32)] * 4,
    )(xT, yT, bT, cT, x, y, qx, s1, s2)

    return (res, t1[0, 0], t2[0, 0], t3)


# speedup vs baseline: 1.6991x; 1.1119x over previous
import jax
import jax.numpy as jnp
from jax.experimental import pallas as pl
from jax.experimental.pallas import tpu as pltpu

N = 4096
M = 4096
BM = 256
GRID = M // BM


def _body(x_ref, y_ref, b_ref, c_ref, iy_ref, xb_ref, yb_ref,
          Q_ref, A_ref, AT_ref,
          res_ref, t1_ref, t2_ref, t3_ref, acc_ref):
    i = pl.program_id(0)

    xT = x_ref[...]
    yT = y_ref[...]
    b_blk = b_ref[...]
    c_blk = c_ref[...]
    iy_blk = iy_ref[...]
    x_blk = xb_ref[...]
    y_blk = yb_ref[...]

    Ax = jnp.sum(A_ref[...] * xT, axis=1, keepdims=True)
    part1 = Ax - b_blk
    part1 = part1 + iy_blk * jnp.maximum(-part1, 0.0)
    s1 = jnp.sum(part1 * part1)

    Qx = jnp.sum(Q_ref[...] * xT, axis=1, keepdims=True)
    ATy = jnp.sum(AT_ref[...] * yT, axis=1, keepdims=True)
    d = Qx + ATy + c_blk
    s2 = jnp.sum(d * d)

    squad = jnp.sum(x_blk * Qx)
    slin = jnp.sum(c_blk * x_blk)
    svio = jnp.sum(b_blk * y_blk)
    sb2 = jnp.sum(b_blk * b_blk)
    sc2 = jnp.sum(c_blk * c_blk)

    parts = (s1, s2, squad, slin, svio, sb2, sc2)

    @pl.when(i == 0)
    def _init():
        for k, v in enumerate(parts):
            acc_ref[k] = v

    @pl.when(i != 0)
    def _accum():
        for k, v in enumerate(parts):
            acc_ref[k] = acc_ref[k] + v

    @pl.when(i == GRID - 1)
    def _fini():
        t1 = jnp.sqrt(acc_ref[0]) / (0.0001 + jnp.sqrt(acc_ref[5]))
        t2 = jnp.sqrt(acc_ref[1]) / (0.0001 + jnp.sqrt(acc_ref[6]))
        t3 = jnp.abs(acc_ref[2] + acc_ref[3] + acc_ref[4])
        t1_ref[0, 0] = t1
        t2_ref[0, 0] = t2
        t3_ref[0, 0] = t3
        res_ref[0, 0] = t1 + t2 + t3


def kernel(Q, A, AT, b, c, x, y, Iy, il, iu, l, u):
    b2 = b[:, None]
    c2 = c[:, None]
    iy2 = Iy[:, None]
    xT = x.T
    yT = y.T

    out_shapes = [jax.ShapeDtypeStruct((1, 1), jnp.float32)] * 4
    full_vec = pl.BlockSpec((1, N), lambda i: (0, 0))
    blk_vec = pl.BlockSpec((BM, 1), lambda i: (i, 0))
    row_blk = pl.BlockSpec((BM, N), lambda i: (i, 0))
    scalar_out = pl.BlockSpec((1, 1), lambda i: (0, 0), memory_space=pltpu.SMEM)

    res, t1, t2, t3 = pl.pallas_call(
        _body,
        grid=(GRID,),
        in_specs=[full_vec, full_vec, blk_vec, blk_vec, blk_vec, blk_vec,
                  blk_vec, row_blk, row_blk, row_blk],
        out_specs=[scalar_out] * 4,
        out_shape=out_shapes,
        scratch_shapes=[pltpu.SMEM((7,), jnp.float32)],
    )(xT, yT, b2, c2, iy2, x, y, Q, A, AT)

    return (res, t1[0, 0], t2[0, 0], t3)
